# Initial kernel scaffold; baseline (speedup 1.0000x reference)
import functools

import jax
import jax.numpy as jnp
from jax import lax
from jax.experimental import pallas as pl
from jax.experimental.pallas import tpu as pltpu
from jax.experimental.pallas import tpu_sc as plsc

N_NODES = 100000
N_EDGES = 3200000
C = 23

NC = 2
NS = 16
NP = 102400
ROWS_PER_TILE = NP // NS
EDGES_PER_TILE_AGG = N_EDGES // NS
EDGES_PER_TILE_DEG = N_EDGES // (NC * NS)
B_AGG = 2000
B_DEG = 5000

_sc_mesh = plsc.VectorSubcoreMesh(
    core_axis_name="c", subcore_axis_name="s", num_cores=NC, num_subcores=NS
)


@functools.partial(
    pl.kernel,
    out_type=jax.ShapeDtypeStruct((NC, NP), jnp.float32),
    mesh=_sc_mesh,
    scratch_types=[
        pltpu.VMEM((B_DEG,), jnp.int32),
        pltpu.VMEM((B_DEG,), jnp.float32),
        pltpu.VMEM_SHARED((NP,), jnp.float32),
    ],
)
def _sc_degree(dst_hbm, ones_hbm, zeros_hbm, out_hbm, idx_v, ones_v, deg_sp):
    c = lax.axis_index("c")
    s = lax.axis_index("s")
    row0 = s * ROWS_PER_TILE
    pltpu.sync_copy(zeros_hbm, deg_sp.at[pl.ds(row0, ROWS_PER_TILE)])
    pltpu.sync_copy(ones_hbm, ones_v)
    plsc.subcore_barrier()
    base = (c * NS + s) * EDGES_PER_TILE_DEG

    def body(i, carry):
        off = base + i * B_DEG
        pltpu.sync_copy(dst_hbm.at[pl.ds(off, B_DEG)], idx_v)
        pltpu.sync_copy(ones_v, deg_sp.at[idx_v], add=True)
        return carry

    lax.fori_loop(0, EDGES_PER_TILE_DEG // B_DEG, body, 0)
    plsc.subcore_barrier()
    pltpu.sync_copy(
        deg_sp.at[pl.ds(row0, ROWS_PER_TILE)],
        out_hbm.at[c, pl.ds(row0, ROWS_PER_TILE)],
    )


@functools.partial(
    pl.kernel,
    out_type=(
        jax.ShapeDtypeStruct((NP, 16), jnp.float32),
        jax.ShapeDtypeStruct((NP, 16), jnp.float32),
    ),
    mesh=_sc_mesh,
    scratch_types=[
        pltpu.VMEM((B_AGG,), jnp.int32),
        pltpu.VMEM((B_AGG,), jnp.int32),
        pltpu.VMEM((B_AGG, 16), jnp.float32),
        pltpu.SemaphoreType.DMA,
        pltpu.VMEM_SHARED((NP, 16), jnp.float32),
    ],
)
def _sc_aggregate(src_hbm, dst_hbm, ga_hbm, gb_hbm, zeros_hbm,
                  outa_hbm, outb_hbm, src_v, dst_v, rows_v, sem, acc_sp):
    c = lax.axis_index("c")
    s = lax.axis_index("s")
    row0 = s * ROWS_PER_TILE
    pltpu.sync_copy(zeros_hbm, acc_sp.at[pl.ds(row0, ROWS_PER_TILE)])
    plsc.subcore_barrier()
    base = s * EDGES_PER_TILE_AGG

    def body(i, carry):
        off = base + i * B_AGG
        pltpu.sync_copy(src_hbm.at[pl.ds(off, B_AGG)], src_v)
        pltpu.sync_copy(dst_hbm.at[pl.ds(off, B_AGG)], dst_v)

        @pl.when(c == 0)
        def _():
            pltpu.async_copy(ga_hbm.at[src_v], rows_v, sem).wait()

        @pl.when(c == 1)
        def _():
            pltpu.async_copy(gb_hbm.at[src_v], rows_v, sem).wait()

        pltpu.sync_copy(rows_v, acc_sp.at[dst_v], add=True)
        return carry

    lax.fori_loop(0, EDGES_PER_TILE_AGG // B_AGG, body, 0)
    plsc.subcore_barrier()

    @pl.when(c == 0)
    def _():
        pltpu.sync_copy(acc_sp.at[pl.ds(row0, ROWS_PER_TILE)],
                        outa_hbm.at[pl.ds(row0, ROWS_PER_TILE)])

    @pl.when(c == 1)
    def _():
        pltpu.sync_copy(acc_sp.at[pl.ds(row0, ROWS_PER_TILE)],
                        outb_hbm.at[pl.ds(row0, ROWS_PER_TILE)])


BLK = 2048


def _prep_body(x_ref, d2_ref, dinv_ref, ga_ref, gb_ref):
    deg = d2_ref[0, :] + d2_ref[1, :] + 1.0
    dinv = lax.rsqrt(deg)
    dinv_ref[...] = dinv
    g = x_ref[...] * dinv[:, None]
    ga_ref[...] = g[:, :16]
    gb_ref[...] = jnp.concatenate(
        [g[:, 16:], jnp.zeros((BLK, 16 - (C - 16)), jnp.float32)], axis=1
    )


_prep_call = pl.pallas_call(
    _prep_body,
    grid=(NP // BLK,),
    in_specs=[
        pl.BlockSpec((BLK, C), lambda i: (i, 0)),
        pl.BlockSpec((NC, BLK), lambda i: (0, i)),
    ],
    out_specs=[
        pl.BlockSpec((BLK,), lambda i: (i,)),
        pl.BlockSpec((BLK, 16), lambda i: (i, 0)),
        pl.BlockSpec((BLK, 16), lambda i: (i, 0)),
    ],
    out_shape=[
        jax.ShapeDtypeStruct((NP,), jnp.float32),
        jax.ShapeDtypeStruct((NP, 16), jnp.float32),
        jax.ShapeDtypeStruct((NP, 16), jnp.float32),
    ],
)


def _mlp_body(x_ref, dinv_ref, aa_ref, ab_ref, wf_ref, b1_ref, w2_ref,
              b2_ref, o_ref):
    dinv = dinv_ref[...]
    acc = jnp.concatenate([aa_ref[...], ab_ref[:, : C - 16]], axis=1)
    x = x_ref[...]
    z = acc * dinv[:, None] + x * (dinv * dinv)[:, None]
    h = jnp.maximum(jnp.dot(z, wf_ref[...]) + b1_ref[...][None, :], 0.0)
    o_ref[...] = x + jnp.dot(h, w2_ref[...]) + b2_ref[...][None, :]


_mlp_call = pl.pallas_call(
    _mlp_body,
    grid=(NP // BLK,),
    in_specs=[
        pl.BlockSpec((BLK, C), lambda i: (i, 0)),
        pl.BlockSpec((BLK,), lambda i: (i,)),
        pl.BlockSpec((BLK, 16), lambda i: (i, 0)),
        pl.BlockSpec((BLK, 16), lambda i: (i, 0)),
        pl.BlockSpec((C, 32), lambda i: (0, 0)),
        pl.BlockSpec((32,), lambda i: (0,)),
        pl.BlockSpec((32, C), lambda i: (0, 0)),
        pl.BlockSpec((C,), lambda i: (0,)),
    ],
    out_specs=pl.BlockSpec((BLK, C), lambda i: (i, 0)),
    out_shape=jax.ShapeDtypeStruct((NP, C), jnp.float32),
)


def kernel(x, edge_index, W_gcn, W1, b1, W2, b2):
    ei32 = edge_index.astype(jnp.int32)
    src, dst = ei32[0], ei32[1]
    Wf = W_gcn @ W1
    zeros1 = jnp.zeros((ROWS_PER_TILE,), jnp.float32)
    zeros2 = jnp.zeros((ROWS_PER_TILE, 16), jnp.float32)
    ones = jnp.ones((B_DEG,), jnp.float32)
    x_pad = jnp.pad(x, ((0, NP - N_NODES), (0, 0)))

    deg2 = _sc_degree(dst, ones, zeros1)
    dinv, ga, gb = _prep_call(x_pad, deg2)
    acca, accb = _sc_aggregate(src, dst, ga, gb, zeros2)
    out_pad = _mlp_call(x_pad, dinv, acca, accb, Wf, b1, W2, b2)
    return out_pad[:N_NODES]


# speedup vs baseline: 58.9352x; 58.9352x over previous
import functools

import jax
import jax.numpy as jnp
from jax import lax
from jax.experimental import pallas as pl
from jax.experimental.pallas import tpu as pltpu
from jax.experimental.pallas import tpu_sc as plsc

N_NODES = 100000
N_EDGES = 3200000
C = 23

NC = 2
NS = 16
NP = 102400
ROWS_PER_TILE = NP // NS
EDGES_PER_TILE_AGG = N_EDGES // NS
EDGES_PER_TILE_DEG = N_EDGES // (NC * NS)
B_AGG = 1000
B_DEG = 5000

_sc_mesh = plsc.VectorSubcoreMesh(
    core_axis_name="c", subcore_axis_name="s", num_cores=NC, num_subcores=NS
)


@functools.partial(
    pl.kernel,
    out_type=jax.ShapeDtypeStruct((NC, NP), jnp.float32),
    mesh=_sc_mesh,
    scratch_types=[
        pltpu.VMEM((B_DEG,), jnp.int32),
        pltpu.VMEM((B_DEG,), jnp.float32),
        pltpu.VMEM_SHARED((NP,), jnp.float32),
    ],
)
def _sc_degree(dst_hbm, ones_hbm, zeros_hbm, out_hbm, idx_v, ones_v, deg_sp):
    c = lax.axis_index("c")
    s = lax.axis_index("s")
    row0 = s * ROWS_PER_TILE
    pltpu.sync_copy(zeros_hbm, deg_sp.at[pl.ds(row0, ROWS_PER_TILE)])
    pltpu.sync_copy(ones_hbm, ones_v)
    plsc.subcore_barrier()
    base = (c * NS + s) * EDGES_PER_TILE_DEG

    def body(i, carry):
        off = base + i * B_DEG
        pltpu.sync_copy(dst_hbm.at[pl.ds(off, B_DEG)], idx_v)
        pltpu.sync_copy(ones_v, deg_sp.at[idx_v], add=True)
        return carry

    lax.fori_loop(0, EDGES_PER_TILE_DEG // B_DEG, body, 0)
    plsc.subcore_barrier()
    pltpu.sync_copy(
        deg_sp.at[pl.ds(row0, ROWS_PER_TILE)],
        out_hbm.at[c, pl.ds(row0, ROWS_PER_TILE)],
    )


@functools.partial(
    pl.kernel,
    out_type=(
        jax.ShapeDtypeStruct((NP, 16), jnp.float32),
        jax.ShapeDtypeStruct((NP, 16), jnp.float32),
    ),
    mesh=_sc_mesh,
    scratch_types=[
        pltpu.VMEM((B_AGG,), jnp.int32),
        pltpu.VMEM((B_AGG,), jnp.int32),
        pltpu.VMEM((B_AGG, 16), jnp.float32),
        pltpu.SemaphoreType.DMA,
        pltpu.VMEM_SHARED((NP, 16), jnp.float32),
    ],
    compiler_params=pltpu.CompilerParams(use_tc_tiling_on_sc=False),
)
def _sc_aggregate(src_hbm, dst_hbm, ga_hbm, gb_hbm, zeros_hbm,
                  outa_hbm, outb_hbm, src_v, dst_v, rows_v, sem, acc_sp):
    c = lax.axis_index("c")
    s = lax.axis_index("s")
    row0 = s * ROWS_PER_TILE
    pltpu.sync_copy(zeros_hbm, acc_sp.at[pl.ds(row0, ROWS_PER_TILE)])
    plsc.subcore_barrier()
    base = s * EDGES_PER_TILE_AGG

    def body(i, carry):
        off = base + i * B_AGG
        pltpu.sync_copy(src_hbm.at[pl.ds(off, B_AGG)], src_v)
        pltpu.sync_copy(dst_hbm.at[pl.ds(off, B_AGG)], dst_v)

        @pl.when(c == 0)
        def _():
            pltpu.async_copy(ga_hbm.at[src_v], rows_v, sem).wait()

        @pl.when(c == 1)
        def _():
            pltpu.async_copy(gb_hbm.at[src_v], rows_v, sem).wait()

        pltpu.sync_copy(rows_v, acc_sp.at[dst_v], add=True)
        return carry

    lax.fori_loop(0, EDGES_PER_TILE_AGG // B_AGG, body, 0)
    plsc.subcore_barrier()

    @pl.when(c == 0)
    def _():
        pltpu.sync_copy(acc_sp.at[pl.ds(row0, ROWS_PER_TILE)],
                        outa_hbm.at[pl.ds(row0, ROWS_PER_TILE)])

    @pl.when(c == 1)
    def _():
        pltpu.sync_copy(acc_sp.at[pl.ds(row0, ROWS_PER_TILE)],
                        outb_hbm.at[pl.ds(row0, ROWS_PER_TILE)])


BLK = 2048


def _prep_body(x_ref, d2_ref, dinv_ref, ga_ref, gb_ref):
    deg = d2_ref[0, :] + d2_ref[1, :] + 1.0
    dinv = lax.rsqrt(deg)
    dinv_ref[...] = dinv
    g = x_ref[...] * dinv[:, None]
    ga_ref[...] = g[:, :16]
    gb_ref[...] = jnp.concatenate(
        [g[:, 16:], jnp.zeros((BLK, 16 - (C - 16)), jnp.float32)], axis=1
    )


_prep_call = pl.pallas_call(
    _prep_body,
    grid=(NP // BLK,),
    in_specs=[
        pl.BlockSpec((BLK, C), lambda i: (i, 0)),
        pl.BlockSpec((NC, BLK), lambda i: (0, i)),
    ],
    out_specs=[
        pl.BlockSpec((BLK,), lambda i: (i,)),
        pl.BlockSpec((BLK, 16), lambda i: (i, 0)),
        pl.BlockSpec((BLK, 16), lambda i: (i, 0)),
    ],
    out_shape=[
        jax.ShapeDtypeStruct((NP,), jnp.float32),
        jax.ShapeDtypeStruct((NP, 16), jnp.float32),
        jax.ShapeDtypeStruct((NP, 16), jnp.float32),
    ],
)


def _mlp_body(x_ref, dinv_ref, aa_ref, ab_ref, wf_ref, b1_ref, w2_ref,
              b2_ref, o_ref):
    dinv = dinv_ref[...]
    acc = jnp.concatenate([aa_ref[...], ab_ref[:, : C - 16]], axis=1)
    x = x_ref[...]
    z = acc * dinv[:, None] + x * (dinv * dinv)[:, None]
    h = jnp.maximum(jnp.dot(z, wf_ref[...]) + b1_ref[...][None, :], 0.0)
    o_ref[...] = x + jnp.dot(h, w2_ref[...]) + b2_ref[...][None, :]


_mlp_call = pl.pallas_call(
    _mlp_body,
    grid=(NP // BLK,),
    in_specs=[
        pl.BlockSpec((BLK, C), lambda i: (i, 0)),
        pl.BlockSpec((BLK,), lambda i: (i,)),
        pl.BlockSpec((BLK, 16), lambda i: (i, 0)),
        pl.BlockSpec((BLK, 16), lambda i: (i, 0)),
        pl.BlockSpec((C, 32), lambda i: (0, 0)),
        pl.BlockSpec((32,), lambda i: (0,)),
        pl.BlockSpec((32, C), lambda i: (0, 0)),
        pl.BlockSpec((C,), lambda i: (0,)),
    ],
    out_specs=pl.BlockSpec((BLK, C), lambda i: (i, 0)),
    out_shape=jax.ShapeDtypeStruct((NP, C), jnp.float32),
)


def kernel(x, edge_index, W_gcn, W1, b1, W2, b2):
    ei32 = edge_index.astype(jnp.int32)
    src, dst = ei32[0], ei32[1]
    Wf = W_gcn @ W1
    zeros1 = jnp.zeros((ROWS_PER_TILE,), jnp.float32)
    zeros2 = jnp.zeros((ROWS_PER_TILE, 16), jnp.float32)
    ones = jnp.ones((B_DEG,), jnp.float32)
    x_pad = jnp.pad(x, ((0, NP - N_NODES), (0, 0)))

    deg2 = _sc_degree(dst, ones, zeros1)
    dinv, ga, gb = _prep_call(x_pad, deg2)
    acca, accb = _sc_aggregate(src, dst, ga, gb, zeros2)
    out_pad = _mlp_call(x_pad, dinv, acca, accb, Wf, b1, W2, b2)
    return out_pad[:N_NODES]


# speedup vs baseline: 81.4098x; 1.3813x over previous
import functools

import jax
import jax.numpy as jnp
from jax import lax
from jax.experimental import pallas as pl
from jax.experimental.pallas import tpu as pltpu
from jax.experimental.pallas import tpu_sc as plsc

N_NODES = 100000
N_EDGES = 3200000
C = 23

NC = 2
NS = 16
NP = 100864
ROWS_PER_TILE = NP // NS
EDGES_PER_TILE_AGG = N_EDGES // NS
EDGES_PER_TILE_DEG = N_EDGES // (NC * NS)
B_AGG = 800
N_CHUNKS = EDGES_PER_TILE_AGG // B_AGG
B_DEG = 5000

_sc_mesh = plsc.VectorSubcoreMesh(
    core_axis_name="c", subcore_axis_name="s", num_cores=NC, num_subcores=NS
)


@functools.partial(
    pl.kernel,
    out_type=jax.ShapeDtypeStruct((NC, NP), jnp.float32),
    mesh=_sc_mesh,
    scratch_types=[
        pltpu.VMEM((B_DEG,), jnp.int32),
        pltpu.VMEM((B_DEG,), jnp.float32),
        pltpu.VMEM_SHARED((NP,), jnp.float32),
    ],
    compiler_params=pltpu.CompilerParams(use_tc_tiling_on_sc=False),
)
def _sc_degree(edge_hbm, ones_hbm, zeros_hbm, out_hbm, idx_v, ones_v, deg_sp):
    c = lax.axis_index("c")
    s = lax.axis_index("s")
    row0 = s * ROWS_PER_TILE
    pltpu.sync_copy(zeros_hbm, deg_sp.at[pl.ds(row0, ROWS_PER_TILE)])
    pltpu.sync_copy(ones_hbm, ones_v)
    plsc.subcore_barrier()
    base = (c * NS + s) * EDGES_PER_TILE_DEG

    def body(i, carry):
        off = base + i * B_DEG
        pltpu.sync_copy(edge_hbm.at[1, pl.ds(off, B_DEG)], idx_v)
        pltpu.sync_copy(ones_v, deg_sp.at[idx_v], add=True)
        return carry

    lax.fori_loop(0, EDGES_PER_TILE_DEG // B_DEG, body, 0)
    plsc.subcore_barrier()
    pltpu.sync_copy(
        deg_sp.at[pl.ds(row0, ROWS_PER_TILE)],
        out_hbm.at[c, pl.ds(row0, ROWS_PER_TILE)],
    )


@functools.partial(
    pl.kernel,
    out_type=(
        jax.ShapeDtypeStruct((NP, 16), jnp.float32),
        jax.ShapeDtypeStruct((NP, 16), jnp.float32),
    ),
    mesh=_sc_mesh,
    scratch_types=[
        pltpu.VMEM((2, B_AGG), jnp.int32),
        pltpu.VMEM((2, B_AGG), jnp.int32),
        pltpu.VMEM((B_AGG, 16), jnp.float32),
        pltpu.VMEM((B_AGG, 16), jnp.float32),
        pltpu.SemaphoreType.DMA,
        pltpu.SemaphoreType.DMA,
        pltpu.VMEM_SHARED((NP, 16), jnp.float32),
    ],
    compiler_params=pltpu.CompilerParams(use_tc_tiling_on_sc=False),
)
def _sc_aggregate(edge_hbm, ga_hbm, gb_hbm, zeros_hbm,
                  outa_hbm, outb_hbm,
                  src_v, dst_v, rows0_v, rows1_v, sem0, sem1, acc_sp):
    c = lax.axis_index("c")
    s = lax.axis_index("s")
    row0 = s * ROWS_PER_TILE
    pltpu.sync_copy(zeros_hbm, acc_sp.at[pl.ds(row0, ROWS_PER_TILE)])
    plsc.subcore_barrier()
    base = s * EDGES_PER_TILE_AGG
    rows = (rows0_v, rows1_v)
    sems = (sem0, sem1)

    def load_and_fire(i, p):
        off = base + i * B_AGG
        pltpu.sync_copy(edge_hbm.at[0, pl.ds(off, B_AGG)], src_v.at[p])
        pltpu.sync_copy(edge_hbm.at[1, pl.ds(off, B_AGG)], dst_v.at[p])

        @pl.when(c == 0)
        def _():
            pltpu.async_copy(ga_hbm.at[src_v.at[p]], rows[p], sems[p])

        @pl.when(c == 1)
        def _():
            pltpu.async_copy(gb_hbm.at[src_v.at[p]], rows[p], sems[p])

    load_and_fire(0, 0)

    def body(j, carry):
        for p in (0, 1):
            i = 2 * j + p

            @pl.when(i < N_CHUNKS - 1)
            def _():
                load_and_fire(i + 1, 1 - p)

            pltpu.make_async_copy(ga_hbm.at[src_v.at[p]], rows[p],
                                  sems[p]).wait()
            pltpu.sync_copy(rows[p], acc_sp.at[dst_v.at[p]], add=True)
        return carry

    lax.fori_loop(0, N_CHUNKS // 2, body, 0)
    plsc.subcore_barrier()

    @pl.when(c == 0)
    def _():
        pltpu.sync_copy(acc_sp.at[pl.ds(row0, ROWS_PER_TILE)],
                        outa_hbm.at[pl.ds(row0, ROWS_PER_TILE)])

    @pl.when(c == 1)
    def _():
        pltpu.sync_copy(acc_sp.at[pl.ds(row0, ROWS_PER_TILE)],
                        outb_hbm.at[pl.ds(row0, ROWS_PER_TILE)])


BLK = 2048
N_BLKS = -(-N_NODES // BLK)


def _prep_body(x_ref, d2_ref, dinv_ref, ga_ref, gb_ref):
    deg = d2_ref[0, :] + d2_ref[1, :] + 1.0
    dinv = lax.rsqrt(deg)
    dinv_ref[...] = dinv
    g = x_ref[...] * dinv[:, None]
    ga_ref[...] = g[:, :16]
    gb_ref[...] = jnp.concatenate(
        [g[:, 16:], jnp.zeros((BLK, 16 - (C - 16)), jnp.float32)], axis=1
    )


_prep_call = pl.pallas_call(
    _prep_body,
    grid=(N_BLKS,),
    in_specs=[
        pl.BlockSpec((BLK, C), lambda i: (i, 0)),
        pl.BlockSpec((NC, BLK), lambda i: (0, i)),
    ],
    out_specs=[
        pl.BlockSpec((BLK,), lambda i: (i,)),
        pl.BlockSpec((BLK, 16), lambda i: (i, 0)),
        pl.BlockSpec((BLK, 16), lambda i: (i, 0)),
    ],
    out_shape=[
        jax.ShapeDtypeStruct((N_NODES,), jnp.float32),
        jax.ShapeDtypeStruct((NP, 16), jnp.float32),
        jax.ShapeDtypeStruct((NP, 16), jnp.float32),
    ],
)


def _mlp_body(x_ref, dinv_ref, aa_ref, ab_ref, wf_ref, b1_ref, w2_ref,
              b2_ref, o_ref):
    dinv = dinv_ref[...]
    acc = jnp.concatenate([aa_ref[...], ab_ref[:, : C - 16]], axis=1)
    x = x_ref[...]
    z = acc * dinv[:, None] + x * (dinv * dinv)[:, None]
    h = jnp.maximum(jnp.dot(z, wf_ref[...]) + b1_ref[...][None, :], 0.0)
    o_ref[...] = x + jnp.dot(h, w2_ref[...]) + b2_ref[...][None, :]


_mlp_call = pl.pallas_call(
    _mlp_body,
    grid=(N_BLKS,),
    in_specs=[
        pl.BlockSpec((BLK, C), lambda i: (i, 0)),
        pl.BlockSpec((BLK,), lambda i: (i,)),
        pl.BlockSpec((BLK, 16), lambda i: (i, 0)),
        pl.BlockSpec((BLK, 16), lambda i: (i, 0)),
        pl.BlockSpec((C, 32), lambda i: (0, 0)),
        pl.BlockSpec((32,), lambda i: (0,)),
        pl.BlockSpec((32, C), lambda i: (0, 0)),
        pl.BlockSpec((C,), lambda i: (0,)),
    ],
    out_specs=pl.BlockSpec((BLK, C), lambda i: (i, 0)),
    out_shape=jax.ShapeDtypeStruct((N_NODES, C), jnp.float32),
)


def kernel(x, edge_index, W_gcn, W1, b1, W2, b2):
    ei32 = edge_index.astype(jnp.int32)
    Wf = W_gcn @ W1
    zeros1 = jnp.zeros((ROWS_PER_TILE,), jnp.float32)
    zeros2 = jnp.zeros((ROWS_PER_TILE, 16), jnp.float32)
    ones = jnp.ones((B_DEG,), jnp.float32)

    deg2 = _sc_degree(ei32, ones, zeros1)
    dinv, ga, gb = _prep_call(x, deg2)
    acca, accb = _sc_aggregate(ei32, ga, gb, zeros2)
    return _mlp_call(x, dinv, acca, accb, Wf, b1, W2, b2)
